# baseline (speedup 1.0000x reference)
import jax
import jax.numpy as jnp
from jax import lax
from jax.experimental import pallas as pl
from jax.experimental.pallas import tpu as pltpu

_EPS = 1e-5
_NLP = 20
_ROWS = 8


def _f2k(f):
    b = lax.bitcast_convert_type(f, jnp.int32)
    return b ^ (lax.shift_right_arithmetic(b, 31) & jnp.int32(0x7FFFFFFF))


def _bsearch(keys, predn, iters=19, nprobe=3):
    lo = jnp.min(keys, axis=1, keepdims=True) - 1
    hi = jnp.max(keys, axis=1, keepdims=True)

    def body(_, c):
        lo, hi = c
        ln = hi - lo
        q1 = lax.shift_right_logical(ln, 1)
        q2 = lax.shift_right_logical(ln, 2)
        if nprobe == 3:
            offs = [q2, q1, q1 + q2]
        else:
            q3 = lax.shift_right_logical(ln, 3)
            offs = [q3, q2, q2 + q3, q1, q1 + q3, q1 + q2, q1 + q2 + q3]
        ms = [lo + o for o in offs[:nprobe]]
        ps = predn(ms)
        nlo, nhi = lo, hi
        for m, p in zip(reversed(ms), reversed(ps)):
            nhi = jnp.where(p, m, nhi)
        for m, p in zip(ms, ps):
            nlo = jnp.where(p, nlo, m)
        return nlo, nhi

    c = (lo, hi)
    for it in range(iters):
        c = body(it, c)
    return c[1]


def _make_body(V):
    def body(y_ref, g_ref, t_ref, tk_ref, tp_ref, samp_ref, vals_ref, idx_ref):
        y = y_ref[...]
        R, Vp = y.shape
        iota = lax.broadcasted_iota(jnp.int32, (R, Vp), 1)
        temp_raw = t_ref[...]
        tk = tk_ref[...]
        tp = tp_ref[...]

        my = jnp.max(y, axis=1, keepdims=True)
        sh_lse = jnp.log(jnp.sum(jnp.exp(y - my), axis=1, keepdims=True))

        temp = jnp.where(temp_raw < _EPS, jnp.float32(1.0), temp_raw)
        x = y / temp
        xk = _f2k(x)

        keff = jnp.where(tk <= 0, jnp.int32(V), jnp.minimum(tk, jnp.int32(V)))

        def pred_k(ms):
            lim = keff - 1
            return [jnp.sum((xk > m).astype(jnp.int32), axis=1,
                            keepdims=True) <= lim for m in ms]

        kth_key = _bsearch(xk, pred_k)

        mx = jnp.max(x, axis=1, keepdims=True)
        e = jnp.where(xk >= kth_key, jnp.exp(x - mx), jnp.float32(0.0))
        zden = jnp.sum(e, axis=1, keepdims=True)
        mass_lim = tp * zden

        def pred_p(ms):
            z0 = jnp.float32(0.0)
            return [jnp.sum(jnp.where(xk > m, e, z0), axis=1,
                            keepdims=True) <= mass_lim for m in ms]

        cut_key = _bsearch(xk, pred_p)

        z = jnp.where(xk >= cut_key, x + g_ref[...], -jnp.inf)
        mz = jnp.max(z, axis=1, keepdims=True)
        samp = jnp.min(jnp.where(z == mz, iota, jnp.int32(Vp)),
                       axis=1, keepdims=True)

        y_work = y
        tvals, tidxs = [], []
        for _ in range(_NLP):
            m = jnp.max(y_work, axis=1, keepdims=True)
            ix = jnp.min(jnp.where(y_work == m, iota, jnp.int32(Vp)),
                         axis=1, keepdims=True)
            tvals.append(m)
            tidxs.append(ix)
            y_work = jnp.where(iota == ix, -jnp.inf, y_work)

        greedy = tidxs[0]
        sampled = jnp.where(temp_raw < _EPS, greedy, samp)
        ysamp = jnp.sum(jnp.where(iota == sampled, y, jnp.float32(0.0)),
                        axis=1, keepdims=True)

        samp_ref[...] = sampled
        vals_ref[...] = jnp.concatenate(
            [(ysamp - my) - sh_lse] + [(m - my) - sh_lse for m in tvals],
            axis=1)
        idx_ref[...] = jnp.concatenate([sampled] + tidxs, axis=1)

    return body


def kernel(logits, temperature, top_k, top_p):
    B, V = logits.shape
    logits = logits.astype(jnp.float32)
    Vp = ((V + 127) // 128) * 128
    gumbel = jax.random.gumbel(jax.random.key(1234), (B, V), jnp.float32)
    ypad = jnp.pad(logits, ((0, 0), (0, Vp - V)), constant_values=-jnp.inf)
    gpad = jnp.pad(gumbel, ((0, 0), (0, Vp - V)), constant_values=0.0)
    t2 = temperature.astype(jnp.float32).reshape(B, 1)
    tk2 = top_k.astype(jnp.int32).reshape(B, 1)
    tp2 = top_p.astype(jnp.float32).reshape(B, 1)

    R = _ROWS
    grid = (B // R,)
    sampled, vals, idx = pl.pallas_call(
        _make_body(V),
        grid=grid,
        compiler_params=pltpu.CompilerParams(
            dimension_semantics=("parallel",)),
        in_specs=[
            pl.BlockSpec((R, Vp), lambda i: (i, 0)),
            pl.BlockSpec((R, Vp), lambda i: (i, 0)),
            pl.BlockSpec((R, 1), lambda i: (i, 0)),
            pl.BlockSpec((R, 1), lambda i: (i, 0)),
            pl.BlockSpec((R, 1), lambda i: (i, 0)),
        ],
        out_specs=[
            pl.BlockSpec((R, 1), lambda i: (i, 0)),
            pl.BlockSpec((R, _NLP + 1), lambda i: (i, 0)),
            pl.BlockSpec((R, _NLP + 1), lambda i: (i, 0)),
        ],
        out_shape=[
            jax.ShapeDtypeStruct((B, 1), jnp.int32),
            jax.ShapeDtypeStruct((B, _NLP + 1), jnp.float32),
            jax.ShapeDtypeStruct((B, _NLP + 1), jnp.int32),
        ],
    )(ypad, gpad, t2, tk2, tp2)
    return sampled, vals, idx


# speedup vs baseline: 1.0776x; 1.0776x over previous
import jax
import jax.numpy as jnp
from jax import lax
from jax.experimental import pallas as pl
from jax.experimental.pallas import tpu as pltpu

_EPS = 1e-5
_NLP = 20
_ROWS = 8


def _f2k(f):
    b = lax.bitcast_convert_type(f, jnp.int32)
    return b ^ (lax.shift_right_arithmetic(b, 31) & jnp.int32(0x7FFFFFFF))


def _bsearch(keys, predn, nprobe=3, lo=None):
    if lo is None:
        lo = jnp.min(keys, axis=1, keepdims=True) - 1
    hi = jnp.max(keys, axis=1, keepdims=True)

    def body(_, c):
        lo, hi = c
        ln = hi - lo
        q1 = lax.shift_right_logical(ln, 1)
        q2 = lax.shift_right_logical(ln, 2)
        if nprobe == 3:
            offs = [q2, q1, q1 + q2]
        else:
            q3 = lax.shift_right_logical(ln, 3)
            offs = [q3, q2, q2 + q3, q1, q1 + q3, q1 + q2, q1 + q2 + q3]
        ms = [lo + o for o in offs[:nprobe]]
        ps = predn(ms)
        nlo, nhi = lo, hi
        for m, p in zip(reversed(ms), reversed(ps)):
            nhi = jnp.where(p, m, nhi)
        for m, p in zip(ms, ps):
            nlo = jnp.where(p, nlo, m)
        return nlo, nhi

    def cond(c):
        lo, hi = c
        return jnp.any(lax.shift_right_logical(hi - lo, 1) > 0)

    lo, hi = lax.while_loop(cond, lambda c: body(0, c), (lo, hi))
    return hi


def _make_body(V):
    def body(y_ref, g_ref, t_ref, tk_ref, tp_ref, samp_ref, vals_ref, idx_ref):
        y = y_ref[...]
        R, Vp = y.shape
        iota = lax.broadcasted_iota(jnp.int32, (R, Vp), 1)
        temp_raw = t_ref[...]
        tk = tk_ref[...]
        tp = tp_ref[...]

        my = jnp.max(y, axis=1, keepdims=True)
        sh_lse = jnp.log(jnp.sum(jnp.exp(y - my), axis=1, keepdims=True))

        temp = jnp.where(temp_raw < _EPS, jnp.float32(1.0), temp_raw)
        x = y / temp
        xk = _f2k(x)

        keff = jnp.where(tk <= 0, jnp.int32(V), jnp.minimum(tk, jnp.int32(V)))

        def pred_k(ms):
            lim = keff - 1
            return [jnp.sum((xk > m).astype(jnp.int32), axis=1,
                            keepdims=True) <= lim for m in ms]

        kth_key = _bsearch(xk, pred_k)

        mx = jnp.max(x, axis=1, keepdims=True)
        e = jnp.where(xk >= kth_key, jnp.exp(x - mx), jnp.float32(0.0))
        zden = jnp.sum(e, axis=1, keepdims=True)
        mass_lim = tp * zden

        def pred_p(ms):
            z0 = jnp.float32(0.0)
            return [jnp.sum(jnp.where(xk > m, e, z0), axis=1,
                            keepdims=True) <= mass_lim for m in ms]

        cut_key = _bsearch(xk, pred_p, lo=kth_key - 1)

        z = jnp.where(xk >= cut_key, x + g_ref[...], -jnp.inf)
        mz = jnp.max(z, axis=1, keepdims=True)
        samp = jnp.min(jnp.where(z == mz, iota, jnp.int32(Vp)),
                       axis=1, keepdims=True)

        y_work = y
        tvals, tidxs = [], []
        for _ in range(_NLP):
            m = jnp.max(y_work, axis=1, keepdims=True)
            ix = jnp.min(jnp.where(y_work == m, iota, jnp.int32(Vp)),
                         axis=1, keepdims=True)
            tvals.append(m)
            tidxs.append(ix)
            y_work = jnp.where(iota == ix, -jnp.inf, y_work)

        greedy = tidxs[0]
        sampled = jnp.where(temp_raw < _EPS, greedy, samp)
        ysamp = jnp.sum(jnp.where(iota == sampled, y, jnp.float32(0.0)),
                        axis=1, keepdims=True)

        samp_ref[...] = sampled
        vals_ref[...] = jnp.concatenate(
            [(ysamp - my) - sh_lse] + [(m - my) - sh_lse for m in tvals],
            axis=1)
        idx_ref[...] = jnp.concatenate([sampled] + tidxs, axis=1)

    return body


def kernel(logits, temperature, top_k, top_p):
    B, V = logits.shape
    logits = logits.astype(jnp.float32)
    Vp = ((V + 127) // 128) * 128
    gumbel = jax.random.gumbel(jax.random.key(1234), (B, V), jnp.float32)
    ypad = jnp.pad(logits, ((0, 0), (0, Vp - V)), constant_values=-jnp.inf)
    gpad = jnp.pad(gumbel, ((0, 0), (0, Vp - V)), constant_values=0.0)
    t2 = temperature.astype(jnp.float32).reshape(B, 1)
    tk2 = top_k.astype(jnp.int32).reshape(B, 1)
    tp2 = top_p.astype(jnp.float32).reshape(B, 1)

    R = _ROWS
    grid = (B // R,)
    sampled, vals, idx = pl.pallas_call(
        _make_body(V),
        grid=grid,
        compiler_params=pltpu.CompilerParams(
            dimension_semantics=("parallel",)),
        in_specs=[
            pl.BlockSpec((R, Vp), lambda i: (i, 0)),
            pl.BlockSpec((R, Vp), lambda i: (i, 0)),
            pl.BlockSpec((R, 1), lambda i: (i, 0)),
            pl.BlockSpec((R, 1), lambda i: (i, 0)),
            pl.BlockSpec((R, 1), lambda i: (i, 0)),
        ],
        out_specs=[
            pl.BlockSpec((R, 1), lambda i: (i, 0)),
            pl.BlockSpec((R, _NLP + 1), lambda i: (i, 0)),
            pl.BlockSpec((R, _NLP + 1), lambda i: (i, 0)),
        ],
        out_shape=[
            jax.ShapeDtypeStruct((B, 1), jnp.int32),
            jax.ShapeDtypeStruct((B, _NLP + 1), jnp.float32),
            jax.ShapeDtypeStruct((B, _NLP + 1), jnp.int32),
        ],
    )(ypad, gpad, t2, tk2, tp2)
    return sampled, vals, idx
